# baseline (speedup 1.0000x reference)
import functools

import jax
import jax.numpy as jnp
from jax import lax
from jax.experimental import pallas as pl
from jax.experimental.pallas import tpu as pltpu
from jax.experimental.pallas import tpu_sc as plsc

V = 100000
E = 300
EP = 320
B = 4096
L = 200
NCH = EP // 32
NC, NS = 2, 16
NW = NC * NS
SEQ_PER_W = B // NW
NBUF = 4


def _augment(table):
    vb = 2000

    def body(t_ref, o_ref):
        x = t_ref[...]
        flag = (jnp.max(jnp.abs(x), axis=1, keepdims=True) > 0.0)
        flag = flag.astype(jnp.float32)
        pad = jnp.zeros((vb, EP - E - 1), jnp.float32)
        o_ref[...] = jnp.concatenate([x, flag, pad], axis=1).astype(jnp.bfloat16)

    return pl.pallas_call(
        body,
        grid=(V // vb,),
        in_specs=[pl.BlockSpec((vb, E), lambda i: (i, 0))],
        out_specs=pl.BlockSpec((vb, EP), lambda i: (i, 0)),
        out_shape=jax.ShapeDtypeStruct((V, EP), jnp.bfloat16),
    )(table)


def _bag(aug, idx):
    mesh = plsc.VectorSubcoreMesh(core_axis_name="c", subcore_axis_name="s")

    @functools.partial(
        pl.kernel,
        out_type=jax.ShapeDtypeStruct((B, EP), jnp.float32),
        mesh=mesh,
        compiler_params=pltpu.CompilerParams(
            use_tc_tiling_on_sc=False, needs_layout_passes=False
        ),
        scratch_types=[
            pltpu.VMEM((NBUF, L), jnp.int32),
            pltpu.VMEM((NBUF, L, EP), jnp.bfloat16),
            pltpu.VMEM((EP,), jnp.float32),
            pltpu.SemaphoreType.DMA,
            pltpu.SemaphoreType.DMA,
            pltpu.SemaphoreType.DMA,
            pltpu.SemaphoreType.DMA,
        ],
    )
    def k(aug_hbm, idx_hbm, out_hbm, idx_v, rows_v, res_v, sem0, sem1, sem2, sem3):
        wid = lax.axis_index("s") * NC + lax.axis_index("c")
        base = wid * SEQ_PER_W
        sems = (sem0, sem1, sem2, sem3)
        lanes = lax.iota(jnp.int32, 16)
        himask = jnp.full((16,), 0xFFFF0000, jnp.uint32)

        def start(kb, b):
            pltpu.sync_copy(idx_hbm.at[b], idx_v.at[kb])
            pltpu.async_copy(aug_hbm.at[idx_v.at[kb]], rows_v.at[kb], sems[kb])

        def finish(kb, b):
            pltpu.make_async_copy(
                aug_hbm.at[idx_v.at[kb]], rows_v.at[kb], sems[kb]
            ).wait()
            buf = rows_v.at[kb]

            def body(j, accs):
                new = []
                for c in range(NCH):
                    w = plsc.bitcast(buf[j, pl.ds(c * 32, 32)], jnp.uint32)
                    fe = plsc.bitcast(w << 16, jnp.float32)
                    fo = plsc.bitcast(w & himask, jnp.float32)
                    new.append(accs[2 * c] + fe)
                    new.append(accs[2 * c + 1] + fo)
                return tuple(new)

            zero = jnp.zeros((16,), jnp.float32)
            accs = lax.fori_loop(
                0, L, body, tuple(zero for _ in range(2 * NCH)), unroll=2
            )
            cnt = accs[2 * 9][6]
            inv = 1.0 / jnp.maximum(jnp.full((16,), cnt), 1.0)
            for c in range(NCH):
                idx_e = c * 32 + 2 * lanes
                plsc.store_scatter(res_v, [idx_e], accs[2 * c] * inv)
                plsc.store_scatter(res_v, [idx_e + 1], accs[2 * c + 1] * inv)
            pltpu.sync_copy(res_v, out_hbm.at[b])

        for kb in range(NBUF):
            start(kb, base + kb)

        @pl.loop(0, SEQ_PER_W, step=NBUF)
        def _(i):
            for kb in range(NBUF):
                b = base + i + kb
                finish(kb, b)

                @pl.when(i + kb + NBUF < SEQ_PER_W)
                def _():
                    start(kb, b + NBUF)

    return k(aug, idx)


def kernel(input, table):
    idx = input.astype(jnp.int32)
    aug = _augment(table)
    out = _bag(aug, idx)
    return out[:, :E]


# speedup vs baseline: 1.1901x; 1.1901x over previous
import functools

import jax
import jax.numpy as jnp
from jax import lax
from jax.experimental import pallas as pl
from jax.experimental.pallas import tpu as pltpu
from jax.experimental.pallas import tpu_sc as plsc

V = 100000
E = 300
EP = 320
EW = EP // 2
NCH = EW // 16
B = 4096
L = 200
NC, NS = 2, 16
NW = NC * NS
SEQ_PER_W = B // NW
NBUF = 4


def _augment(table):
    vb = 2000

    def body(t_ref, o_ref):
        x = t_ref[...]
        flag = (jnp.max(jnp.abs(x), axis=1, keepdims=True) > 0.0)
        flag = flag.astype(jnp.float32)
        pad = jnp.zeros((vb, EP - E - 1), jnp.float32)
        aug = jnp.concatenate([x, flag, pad], axis=1)
        lo = lax.bitcast_convert_type(
            aug[:, :EW].astype(jnp.bfloat16), jnp.uint16
        ).astype(jnp.uint32)
        hi = lax.bitcast_convert_type(
            aug[:, EW:].astype(jnp.bfloat16), jnp.uint16
        ).astype(jnp.uint32)
        o_ref[...] = lo | (hi << 16)

    return pl.pallas_call(
        body,
        grid=(V // vb,),
        in_specs=[pl.BlockSpec((vb, E), lambda i: (i, 0))],
        out_specs=pl.BlockSpec((vb, EW), lambda i: (i, 0)),
        out_shape=jax.ShapeDtypeStruct((V, EW), jnp.uint32),
    )(table)


def _bag(aug, idx):
    mesh = plsc.VectorSubcoreMesh(core_axis_name="c", subcore_axis_name="s")

    @functools.partial(
        pl.kernel,
        out_type=jax.ShapeDtypeStruct((B, EP), jnp.float32),
        mesh=mesh,
        compiler_params=pltpu.CompilerParams(
            use_tc_tiling_on_sc=False, needs_layout_passes=False
        ),
        scratch_types=[
            pltpu.VMEM((NBUF, L), jnp.int32),
            pltpu.VMEM((NBUF, L, EW), jnp.uint32),
            pltpu.VMEM((EP,), jnp.float32),
            pltpu.SemaphoreType.DMA,
            pltpu.SemaphoreType.DMA,
            pltpu.SemaphoreType.DMA,
            pltpu.SemaphoreType.DMA,
        ],
    )
    def k(aug_hbm, idx_hbm, out_hbm, idx_v, rows_v, res_v, sem0, sem1, sem2, sem3):
        wid = lax.axis_index("s") * NC + lax.axis_index("c")
        base = wid * SEQ_PER_W
        sems = (sem0, sem1, sem2, sem3)
        himask = jnp.full((16,), 0xFFFF0000, jnp.uint32)

        def start(kb, b):
            pltpu.sync_copy(idx_hbm.at[b], idx_v.at[kb])
            pltpu.async_copy(aug_hbm.at[idx_v.at[kb]], rows_v.at[kb], sems[kb])

        def finish(kb, b):
            pltpu.make_async_copy(
                aug_hbm.at[idx_v.at[kb]], rows_v.at[kb], sems[kb]
            ).wait()
            buf = rows_v.at[kb]

            def body(j, accs):
                new = []
                for c in range(NCH):
                    w = buf[j, pl.ds(c * 16, 16)]
                    fe = plsc.bitcast(w << 16, jnp.float32)
                    fo = plsc.bitcast(w & himask, jnp.float32)
                    new.append(accs[2 * c] + fe)
                    new.append(accs[2 * c + 1] + fo)
                return tuple(new)

            zero = jnp.zeros((16,), jnp.float32)
            accs = lax.fori_loop(
                0, L, body, tuple(zero for _ in range(2 * NCH)), unroll=2
            )
            cnt = accs[2 * 8 + 1][12]
            inv = 1.0 / jnp.maximum(jnp.full((16,), cnt), 1.0)
            for c in range(NCH):
                res_v[pl.ds(c * 16, 16)] = accs[2 * c] * inv
                res_v[pl.ds(EW + c * 16, 16)] = accs[2 * c + 1] * inv
            pltpu.sync_copy(res_v, out_hbm.at[b])

        for kb in range(NBUF):
            start(kb, base + kb)

        @pl.loop(0, SEQ_PER_W, step=NBUF)
        def _(i):
            for kb in range(NBUF):
                b = base + i + kb
                finish(kb, b)

                @pl.when(i + kb + NBUF < SEQ_PER_W)
                def _():
                    start(kb, b + NBUF)

    return k(aug, idx)


def kernel(input, table):
    idx = input.astype(jnp.int32)
    aug = _augment(table)
    out = _bag(aug, idx)
    return out[:, :E]


# speedup vs baseline: 1.2843x; 1.0792x over previous
import functools

import jax
import jax.numpy as jnp
from jax import lax
from jax.experimental import pallas as pl
from jax.experimental.pallas import tpu as pltpu
from jax.experimental.pallas import tpu_sc as plsc

V = 100000
E = 300
EP = 320
EW = EP // 2
NCH = EW // 16
B = 4096
L = 200
NC, NS = 2, 16
NW = NC * NS
SEQ_PER_W = B // NW
NBUF = 4


def _augment(table):
    vb = 2000

    def body(t_ref, o_ref):
        x = t_ref[...]
        flag = (jnp.max(jnp.abs(x), axis=1, keepdims=True) > 0.0)
        flag = flag.astype(jnp.float32)
        pad = jnp.zeros((vb, EP - E - 1), jnp.float32)
        aug = jnp.concatenate([x, flag, pad], axis=1)
        lo = lax.bitcast_convert_type(
            aug[:, :EW].astype(jnp.bfloat16), jnp.uint16
        ).astype(jnp.uint32)
        hi = lax.bitcast_convert_type(
            aug[:, EW:].astype(jnp.bfloat16), jnp.uint16
        ).astype(jnp.uint32)
        o_ref[...] = lo | (hi << 16)

    return pl.pallas_call(
        body,
        grid=(V // vb,),
        in_specs=[pl.BlockSpec((vb, E), lambda i: (i, 0))],
        out_specs=pl.BlockSpec((vb, EW), lambda i: (i, 0)),
        out_shape=jax.ShapeDtypeStruct((V, EW), jnp.uint32),
    )(table)


def _bag(aug, idx):
    mesh = plsc.VectorSubcoreMesh(core_axis_name="c", subcore_axis_name="s")

    @functools.partial(
        pl.kernel,
        out_type=jax.ShapeDtypeStruct((B, EP), jnp.float32),
        mesh=mesh,
        compiler_params=pltpu.CompilerParams(
            use_tc_tiling_on_sc=False, needs_layout_passes=False
        ),
        scratch_types=[
            pltpu.VMEM((NBUF, L), jnp.int32),
            pltpu.VMEM((NBUF, L, EW), jnp.uint32),
            pltpu.VMEM((NBUF, EP), jnp.float32),
            pltpu.SemaphoreType.DMA,
            pltpu.SemaphoreType.DMA,
            pltpu.SemaphoreType.DMA,
            pltpu.SemaphoreType.DMA,
            pltpu.SemaphoreType.DMA,
            pltpu.SemaphoreType.DMA,
            pltpu.SemaphoreType.DMA,
            pltpu.SemaphoreType.DMA,
        ],
    )
    def k(aug_hbm, idx_hbm, out_hbm, idx_v, rows_v, res_v,
          sem0, sem1, sem2, sem3, osem0, osem1, osem2, osem3):
        wid = lax.axis_index("s") * NC + lax.axis_index("c")
        base = wid * SEQ_PER_W
        sems = (sem0, sem1, sem2, sem3)
        osems = (osem0, osem1, osem2, osem3)
        himask = jnp.full((16,), 0xFFFF0000, jnp.uint32)

        def start(kb, b):
            pltpu.sync_copy(idx_hbm.at[b], idx_v.at[kb])
            pltpu.async_copy(aug_hbm.at[idx_v.at[kb]], rows_v.at[kb], sems[kb])

        def finish(kb, b):
            pltpu.make_async_copy(
                aug_hbm.at[idx_v.at[kb]], rows_v.at[kb], sems[kb]
            ).wait()
            buf = rows_v.at[kb]

            def body(jp, accs):
                new = []
                for c in range(NCH):
                    w0 = buf[2 * jp, pl.ds(c * 16, 16)]
                    w1 = buf[2 * jp + 1, pl.ds(c * 16, 16)]
                    s = plsc.bitcast(w0, jnp.bfloat16) + plsc.bitcast(w1, jnp.bfloat16)
                    w = plsc.bitcast(s, jnp.uint32)
                    fe = plsc.bitcast(w << 16, jnp.float32)
                    fo = plsc.bitcast(w & himask, jnp.float32)
                    new.append(accs[2 * c] + fe)
                    new.append(accs[2 * c + 1] + fo)
                return tuple(new)

            zero = jnp.zeros((16,), jnp.float32)
            accs = lax.fori_loop(
                0, L // 2, body, tuple(zero for _ in range(2 * NCH)), unroll=2
            )
            cnt = accs[2 * 8 + 1][12]
            inv = 1.0 / jnp.maximum(jnp.full((16,), cnt), 1.0)

            @pl.when(b - base >= NBUF)
            def _():
                pltpu.make_async_copy(
                    res_v.at[kb], out_hbm.at[b - NBUF], osems[kb]
                ).wait()

            for c in range(NCH):
                res_v[kb, pl.ds(c * 16, 16)] = accs[2 * c] * inv
                res_v[kb, pl.ds(EW + c * 16, 16)] = accs[2 * c + 1] * inv
            pltpu.async_copy(res_v.at[kb], out_hbm.at[b], osems[kb])

        for kb in range(NBUF):
            start(kb, base + kb)

        @pl.loop(0, SEQ_PER_W, step=NBUF)
        def _(i):
            for kb in range(NBUF):
                b = base + i + kb
                finish(kb, b)

                @pl.when(i + kb + NBUF < SEQ_PER_W)
                def _():
                    start(kb, b + NBUF)

        for kb in range(NBUF):
            pltpu.make_async_copy(
                res_v.at[kb], out_hbm.at[base + SEQ_PER_W - NBUF + kb], osems[kb]
            ).wait()

    return k(aug, idx)


def kernel(input, table):
    idx = input.astype(jnp.int32)
    aug = _augment(table)
    out = _bag(aug, idx)
    return out[:, :E]


# speedup vs baseline: 1.3806x; 1.0750x over previous
import functools

import jax
import jax.numpy as jnp
from jax import lax
from jax.experimental import pallas as pl
from jax.experimental.pallas import tpu as pltpu
from jax.experimental.pallas import tpu_sc as plsc

V = 100000
E = 300
EP = 320
EW = EP // 2
EWP = 256
NCH = EW // 16
B = 4096
L = 200
NC, NS = 2, 16
NW = NC * NS
SEQ_PER_W = B // NW
NBUF = 2


def _augment(table):
    vb = 2000

    def body(t_ref, o_ref):
        x = t_ref[...]
        flag = (jnp.max(jnp.abs(x), axis=1, keepdims=True) > 0.0)
        flag = flag.astype(jnp.float32)
        pad = jnp.zeros((vb, EP - E - 1), jnp.float32)
        aug = jnp.concatenate([x, flag, pad], axis=1)
        lo = lax.bitcast_convert_type(
            aug[:, :EW].astype(jnp.bfloat16), jnp.uint16
        ).astype(jnp.uint32)
        hi = lax.bitcast_convert_type(
            aug[:, EW:].astype(jnp.bfloat16), jnp.uint16
        ).astype(jnp.uint32)
        zpad = jnp.zeros((vb, EWP - EW), jnp.uint32)
        o_ref[...] = jnp.concatenate([lo | (hi << 16), zpad], axis=1)

    return pl.pallas_call(
        body,
        grid=(V // vb,),
        in_specs=[pl.BlockSpec((vb, E), lambda i: (i, 0))],
        out_specs=pl.BlockSpec((vb, EWP), lambda i: (i, 0)),
        out_shape=jax.ShapeDtypeStruct((V, EWP), jnp.uint32),
    )(table)


def _bag(aug, idx):
    mesh = plsc.VectorSubcoreMesh(core_axis_name="c", subcore_axis_name="s")

    @functools.partial(
        pl.kernel,
        out_type=jax.ShapeDtypeStruct((B, EP), jnp.float32),
        mesh=mesh,
        compiler_params=pltpu.CompilerParams(
            use_tc_tiling_on_sc=True, needs_layout_passes=False
        ),
        scratch_types=[
            pltpu.VMEM((NBUF, L), jnp.int32),
            pltpu.VMEM((NBUF, L, EWP), jnp.uint32),
            pltpu.VMEM((NBUF, EP), jnp.float32),
            pltpu.SemaphoreType.DMA,
            pltpu.SemaphoreType.DMA,
            pltpu.SemaphoreType.DMA,
            pltpu.SemaphoreType.DMA,
        ],
    )
    def k(aug_hbm, idx_hbm, out_hbm, idx_v, rows_v, res_v,
          sem0, sem1, osem0, osem1):
        wid = lax.axis_index("s") * NC + lax.axis_index("c")
        base = wid * SEQ_PER_W
        sems = (sem0, sem1)
        osems = (osem0, osem1)
        himask = jnp.full((16,), 0xFFFF0000, jnp.uint32)

        def start(kb, b):
            pltpu.sync_copy(idx_hbm.at[b], idx_v.at[kb])
            pltpu.async_copy(
                aug_hbm.at[idx_v.at[kb, pl.ds(0, 128)]],
                rows_v.at[kb, pl.ds(0, 128)], sems[kb])
            pltpu.async_copy(
                aug_hbm.at[idx_v.at[kb, pl.ds(128, 72)]],
                rows_v.at[kb, pl.ds(128, 72)], sems[kb])

        def finish(kb, b):
            pltpu.make_async_copy(
                aug_hbm.at[idx_v.at[kb, pl.ds(0, 128)]],
                rows_v.at[kb, pl.ds(0, 128)], sems[kb]
            ).wait()
            pltpu.make_async_copy(
                aug_hbm.at[idx_v.at[kb, pl.ds(128, 72)]],
                rows_v.at[kb, pl.ds(128, 72)], sems[kb]
            ).wait()
            buf = rows_v.at[kb]

            def body(jp, accs):
                new = []
                for c in range(NCH):
                    w0 = buf[2 * jp, pl.ds(c * 16, 16)]
                    w1 = buf[2 * jp + 1, pl.ds(c * 16, 16)]
                    s = plsc.bitcast(w0, jnp.bfloat16) + plsc.bitcast(w1, jnp.bfloat16)
                    w = plsc.bitcast(s, jnp.uint32)
                    fe = plsc.bitcast(w << 16, jnp.float32)
                    fo = plsc.bitcast(w & himask, jnp.float32)
                    new.append(accs[2 * c] + fe)
                    new.append(accs[2 * c + 1] + fo)
                return tuple(new)

            zero = jnp.zeros((16,), jnp.float32)
            accs = lax.fori_loop(
                0, L // 2, body, tuple(zero for _ in range(2 * NCH)), unroll=2
            )
            cnt = accs[2 * 8 + 1][12]
            inv = 1.0 / jnp.maximum(jnp.full((16,), cnt), 1.0)

            @pl.when(b - base >= NBUF)
            def _():
                pltpu.make_async_copy(
                    res_v.at[kb], out_hbm.at[b - NBUF], osems[kb]
                ).wait()

            for c in range(NCH):
                res_v[kb, pl.ds(c * 16, 16)] = accs[2 * c] * inv
                res_v[kb, pl.ds(EW + c * 16, 16)] = accs[2 * c + 1] * inv
            pltpu.async_copy(res_v.at[kb], out_hbm.at[b], osems[kb])

        for kb in range(NBUF):
            start(kb, base + kb)

        @pl.loop(0, SEQ_PER_W, step=NBUF)
        def _(i):
            for kb in range(NBUF):
                b = base + i + kb
                finish(kb, b)

                @pl.when(i + kb + NBUF < SEQ_PER_W)
                def _():
                    start(kb, b + NBUF)

        for kb in range(NBUF):
            pltpu.make_async_copy(
                res_v.at[kb], out_hbm.at[base + SEQ_PER_W - NBUF + kb], osems[kb]
            ).wait()

    return k(aug, idx)


def kernel(input, table):
    idx = input.astype(jnp.int32)
    aug = _augment(table)
    out = _bag(aug, idx)
    return out[:, :E]


# speedup vs baseline: 1.7485x; 1.2664x over previous
import functools

import jax
import jax.numpy as jnp
from jax import lax
from jax.experimental import pallas as pl
from jax.experimental.pallas import tpu as pltpu
from jax.experimental.pallas import tpu_sc as plsc

V = 100000
E = 300
EP = 320
EW = EP // 2
EWP = 256
NCH = EW // 16
B = 4096
L = 200
NC, NS = 2, 16
NW = NC * NS
SEQ_PER_W = B // NW
NBUF = 2


def _augment(table_t):
    vb = 2048

    def body(t_ref, o_ref):
        x = t_ref[...].T
        flag = (jnp.max(jnp.abs(x), axis=1, keepdims=True) > 0.0)
        flag = flag.astype(jnp.float32)
        pad = jnp.zeros((vb, EP - E - 1), jnp.float32)
        aug = jnp.concatenate([x, flag, pad], axis=1)
        lo = lax.bitcast_convert_type(
            aug[:, :EW].astype(jnp.bfloat16), jnp.uint16
        ).astype(jnp.uint32)
        hi = lax.bitcast_convert_type(
            aug[:, EW:].astype(jnp.bfloat16), jnp.uint16
        ).astype(jnp.uint32)
        zpad = jnp.zeros((vb, EWP - EW), jnp.uint32)
        o_ref[...] = jnp.concatenate([lo | (hi << 16), zpad], axis=1)

    return pl.pallas_call(
        body,
        grid=((V + vb - 1) // vb,),
        in_specs=[pl.BlockSpec((E, vb), lambda i: (0, i))],
        out_specs=pl.BlockSpec((vb, EWP), lambda i: (i, 0)),
        out_shape=jax.ShapeDtypeStruct((V, EWP), jnp.uint32),
    )(table_t)


def _bag(aug, idx):
    mesh = plsc.VectorSubcoreMesh(core_axis_name="c", subcore_axis_name="s")

    @functools.partial(
        pl.kernel,
        out_type=jax.ShapeDtypeStruct((B, EP), jnp.float32),
        mesh=mesh,
        compiler_params=pltpu.CompilerParams(
            use_tc_tiling_on_sc=True, needs_layout_passes=False
        ),
        scratch_types=[
            pltpu.VMEM((NBUF, L), jnp.int32),
            pltpu.VMEM((NBUF, L, EWP), jnp.uint32),
            pltpu.VMEM((NBUF, EP), jnp.float32),
            pltpu.SemaphoreType.DMA,
            pltpu.SemaphoreType.DMA,
            pltpu.SemaphoreType.DMA,
            pltpu.SemaphoreType.DMA,
        ],
    )
    def k(aug_hbm, idx_hbm, out_hbm, idx_v, rows_v, res_v,
          sem0, sem1, osem0, osem1):
        wid = lax.axis_index("s") * NC + lax.axis_index("c")
        base = wid * SEQ_PER_W
        sems = (sem0, sem1)
        osems = (osem0, osem1)
        himask = jnp.full((16,), 0xFFFF0000, jnp.uint32)

        def start(kb, b):
            pltpu.sync_copy(idx_hbm.at[b], idx_v.at[kb])
            pltpu.async_copy(
                aug_hbm.at[idx_v.at[kb, pl.ds(0, 128)]],
                rows_v.at[kb, pl.ds(0, 128)], sems[kb])
            pltpu.async_copy(
                aug_hbm.at[idx_v.at[kb, pl.ds(128, 72)]],
                rows_v.at[kb, pl.ds(128, 72)], sems[kb])

        def finish(kb, b):
            pltpu.make_async_copy(
                aug_hbm.at[idx_v.at[kb, pl.ds(0, 128)]],
                rows_v.at[kb, pl.ds(0, 128)], sems[kb]
            ).wait()
            pltpu.make_async_copy(
                aug_hbm.at[idx_v.at[kb, pl.ds(128, 72)]],
                rows_v.at[kb, pl.ds(128, 72)], sems[kb]
            ).wait()
            buf = rows_v.at[kb]

            def body(jp, accs):
                new = []
                for c in range(NCH):
                    w0 = buf[2 * jp, pl.ds(c * 16, 16)]
                    w1 = buf[2 * jp + 1, pl.ds(c * 16, 16)]
                    s = plsc.bitcast(w0, jnp.bfloat16) + plsc.bitcast(w1, jnp.bfloat16)
                    w = plsc.bitcast(s, jnp.uint32)
                    fe = plsc.bitcast(w << 16, jnp.float32)
                    fo = plsc.bitcast(w & himask, jnp.float32)
                    new.append(accs[2 * c] + fe)
                    new.append(accs[2 * c + 1] + fo)
                return tuple(new)

            zero = jnp.zeros((16,), jnp.float32)
            accs = lax.fori_loop(
                0, L // 2, body, tuple(zero for _ in range(2 * NCH)), unroll=2
            )
            cnt = accs[2 * 8 + 1][12]
            inv = 1.0 / jnp.maximum(jnp.full((16,), cnt), 1.0)

            @pl.when(b - base >= NBUF)
            def _():
                pltpu.make_async_copy(
                    res_v.at[kb], out_hbm.at[b - NBUF], osems[kb]
                ).wait()

            for c in range(NCH):
                res_v[kb, pl.ds(c * 16, 16)] = accs[2 * c] * inv
                res_v[kb, pl.ds(EW + c * 16, 16)] = accs[2 * c + 1] * inv
            pltpu.async_copy(res_v.at[kb], out_hbm.at[b], osems[kb])

        for kb in range(NBUF):
            start(kb, base + kb)

        @pl.loop(0, SEQ_PER_W, step=NBUF)
        def _(i):
            for kb in range(NBUF):
                b = base + i + kb
                finish(kb, b)

                @pl.when(i + kb + NBUF < SEQ_PER_W)
                def _():
                    start(kb, b + NBUF)

        for kb in range(NBUF):
            pltpu.make_async_copy(
                res_v.at[kb], out_hbm.at[base + SEQ_PER_W - NBUF + kb], osems[kb]
            ).wait()

    return k(aug, idx)


def kernel(input, table):
    idx = input.astype(jnp.int32)
    aug = _augment(table.T)
    out = _bag(aug, idx)
    return out[:, :E]
